# baseline (speedup 1.0000x reference)
import functools

import jax
import jax.numpy as jnp
from jax import lax
from jax.experimental import pallas as pl
from jax.experimental.pallas import tpu as pltpu
from jax.experimental.pallas import tpu_sc as plsc

VOC = 1_000_000
DIM = 32
ROWS = 16384
COLS = 26
PAD = 128
NC = 2
NS = 16
NW = NC * NS
BPW = ROWS // NW
CHB = 256
NCHUNK = COLS * (BPW // CHB)
NBUF = 2
PER_W = COLS * BPW
VBLK = 512


@functools.partial(
    pl.pallas_call,
    out_shape=jax.ShapeDtypeStruct((VOC // 4, PAD), jnp.float32),
    grid=((VOC + VBLK - 1) // VBLK,),
    in_specs=[pl.BlockSpec((DIM, VBLK), lambda i: (0, i))],
    out_specs=pl.BlockSpec((VBLK // 4, PAD), lambda i: (i, 0)),
)
def _tc_pack(tt_ref, o_ref):
    t = tt_ref[...].T.reshape(VBLK // 4, 4, DIM)
    o_ref[...] = jnp.concatenate([t[:, q, :] for q in range(4)], axis=1)


@functools.partial(
    pl.kernel,
    out_type=jax.ShapeDtypeStruct((COLS, DIM, ROWS), jnp.float32),
    mesh=plsc.VectorSubcoreMesh(core_axis_name="c", subcore_axis_name="s"),
    scratch_types=(
        [pltpu.VMEM((PER_W,), jnp.int32),
         pltpu.VMEM((PER_W,), jnp.int32)]
        + [pltpu.VMEM((CHB, PAD), jnp.float32) for _ in range(NBUF)]
        + [pltpu.VMEM((DIM, CHB), jnp.float32) for _ in range(NBUF)]
        + [pltpu.SemaphoreType.DMA for _ in range(NBUF)]
        + [pltpu.SemaphoreType.DMA]
    ),
    compiler_params=pltpu.CompilerParams(
        use_tc_tiling_on_sc=True, needs_layout_passes=False),
)
def _sc_gather(xt_hbm, tp_hbm, out_hbm, idx_v, off_v, *bufs):
    wid = lax.axis_index("s") * NC + lax.axis_index("c")
    b0 = wid * BPW

    wide = bufs[:NBUF]
    comp = bufs[NBUF:2 * NBUF]
    sems = bufs[2 * NBUF:3 * NBUF]
    isem = bufs[3 * NBUF]

    for s in range(COLS):
        pltpu.async_copy(
            xt_hbm.at[s, pl.ds(b0, BPW)],
            idx_v.at[pl.ds(s * BPW, BPW)], isem)
    for s in range(COLS):
        pltpu.make_async_copy(
            xt_hbm.at[s, pl.ds(b0, BPW)],
            idx_v.at[pl.ds(s * BPW, BPW)], isem).wait()

    iota = lax.iota(jnp.int32, 16)

    def split(i, carry):
        v = idx_v[pl.ds(i * 16, 16)]
        idx_v[pl.ds(i * 16, 16)] = lax.shift_right_logical(v, 2)
        off_v[pl.ds(i * 16, 16)] = lax.shift_left(
            lax.bitwise_and(v, jnp.int32(3)), 5)
        return carry
    lax.fori_loop(0, PER_W // 16, split, 0)

    def start_gather(c, b):
        pltpu.async_copy(
            tp_hbm.at[idx_v.at[pl.ds(c * CHB, CHB)]], wide[b], sems[b])

    def wait_gather(c, b):
        pltpu.make_async_copy(
            tp_hbm.at[idx_v.at[pl.ds(c * CHB, CHB)]], wide[b],
            sems[b]).wait()

    def extract(c, b):
        def grp(g, carry):
            kvec = g * 16 + iota
            cvec = off_v[pl.ds(c * CHB + g * 16, 16)]
            for d in range(DIM):
                val = plsc.load_gather(wide[b], [kvec, cvec + d])
                comp[b][d, pl.ds(g * 16, 16)] = val
            return carry
        lax.fori_loop(0, CHB // 16, grp, 0)

    def writeback(c, b):
        s = c >> 1
        bb = b0 + (c & 1) * CHB
        pltpu.sync_copy(comp[b], out_hbm.at[s, :, pl.ds(bb, CHB)])

    def step(c, b):
        wait_gather(c, b)
        extract(c, b)
        writeback(c, b)

    for b in range(NBUF):
        start_gather(b, b)

    def body(i, carry):
        for b in range(NBUF):
            c = i * NBUF + b
            step(c, b)
            start_gather(c + NBUF, b)
        return carry

    lax.fori_loop(0, (NCHUNK - NBUF) // NBUF, body, 0)
    for c in range(NCHUNK - NBUF, NCHUNK):
        step(c, c % NBUF)


def kernel(x, table):
    tp = _tc_pack(table.T)
    out = _sc_gather(x.T, tp)
    return out.transpose(2, 0, 1)


# speedup vs baseline: 1.0218x; 1.0218x over previous
import functools

import jax
import jax.numpy as jnp
from jax import lax
from jax.experimental import pallas as pl
from jax.experimental.pallas import tpu as pltpu
from jax.experimental.pallas import tpu_sc as plsc

VOC = 1_000_000
DIM = 32
ROWS = 16384
COLS = 26
PAD = 128
NC = 2
NS = 16
NW = NC * NS
BPW = ROWS // NW
CHB = 256
NCHUNK = COLS * (BPW // CHB)
NBUF = 3
PER_W = COLS * BPW
VBLK = 512


@functools.partial(
    pl.pallas_call,
    out_shape=jax.ShapeDtypeStruct((VOC, PAD), jnp.float32),
    grid=((VOC + VBLK - 1) // VBLK,),
    in_specs=[pl.BlockSpec((DIM, VBLK), lambda i: (0, i))],
    out_specs=pl.BlockSpec((VBLK, PAD), lambda i: (i, 0)),
)
def _tc_pad(tt_ref, o_ref):
    o_ref[:, :DIM] = tt_ref[...].T


@functools.partial(
    pl.kernel,
    out_type=jax.ShapeDtypeStruct((COLS, DIM, ROWS), jnp.float32),
    mesh=plsc.VectorSubcoreMesh(core_axis_name="c", subcore_axis_name="s"),
    scratch_types=(
        [pltpu.VMEM((PER_W,), jnp.int32)]
        + [pltpu.VMEM((CHB, PAD), jnp.float32) for _ in range(NBUF)]
        + [pltpu.VMEM((DIM, CHB), jnp.float32)]
        + [pltpu.SemaphoreType.DMA for _ in range(NBUF)]
        + [pltpu.SemaphoreType.DMA]
    ),
    compiler_params=pltpu.CompilerParams(
        use_tc_tiling_on_sc=True, needs_layout_passes=False,
        disable_bounds_checks=True),
)
def _sc_gather(xt_hbm, tp_hbm, out_hbm, idx_v, *bufs):
    wid = lax.axis_index("s") * NC + lax.axis_index("c")
    b0 = wid * BPW

    wide = bufs[:NBUF]
    comp = bufs[NBUF]
    sems = bufs[NBUF + 1:2 * NBUF + 1]
    isem = bufs[2 * NBUF + 1]

    for s in range(COLS):
        pltpu.async_copy(
            xt_hbm.at[s, pl.ds(b0, BPW)],
            idx_v.at[pl.ds(s * BPW, BPW)], isem)
    for s in range(COLS):
        pltpu.make_async_copy(
            xt_hbm.at[s, pl.ds(b0, BPW)],
            idx_v.at[pl.ds(s * BPW, BPW)], isem).wait()

    iota = lax.iota(jnp.int32, 16)

    def start_gather(c, b):
        pltpu.async_copy(
            tp_hbm.at[idx_v.at[pl.ds(c * CHB, CHB)]], wide[b], sems[b])

    def wait_gather(c, b):
        pltpu.make_async_copy(
            tp_hbm.at[idx_v.at[pl.ds(c * CHB, CHB)]], wide[b],
            sems[b]).wait()

    def extract(b):
        def grp(g, carry):
            kvec = g * 16 + iota
            for d in range(DIM):
                val = plsc.load_gather(
                    wide[b], [kvec, jnp.full((16,), d, jnp.int32)])
                comp[d, pl.ds(g * 16, 16)] = val
            return carry
        lax.fori_loop(0, CHB // 16, grp, 0)

    def writeback(c, b):
        s = c >> 1
        bb = b0 + (c & 1) * CHB
        pltpu.sync_copy(comp, out_hbm.at[s, :, pl.ds(bb, CHB)])

    def step(c, b):
        wait_gather(c, b)
        extract(b)
        writeback(c, b)

    for b in range(NBUF):
        start_gather(b, b)

    def body(i, carry):
        for b in range(NBUF):
            c = i * NBUF + b
            step(c, b)
            start_gather(c + NBUF, b)
        return carry

    steady = (NCHUNK - NBUF) // NBUF
    lax.fori_loop(0, steady, body, 0)
    for c in range(steady * NBUF, NCHUNK):
        step(c, c % NBUF)
        if c + NBUF < NCHUNK:
            start_gather(c + NBUF, c % NBUF)


def kernel(x, table):
    tp = _tc_pad(table.T)
    out = _sc_gather(x.T, tp)
    return out.transpose(2, 0, 1)


# speedup vs baseline: 1.9174x; 1.8764x over previous
import functools

import jax
import jax.numpy as jnp
from jax import lax
from jax.experimental import pallas as pl
from jax.experimental.pallas import tpu as pltpu
from jax.experimental.pallas import tpu_sc as plsc

VOC = 1_000_000
DIM = 32
ROWS = 16384
COLS = 26
PAD = 128
NC = 2
NS = 16
NW = NC * NS
BPW = ROWS // NW
CHB = 256
NCHUNK = COLS * (BPW // CHB)
NBUF = 3
PER_W = COLS * BPW
VBLK = 2048


@functools.partial(
    pl.pallas_call,
    out_shape=jax.ShapeDtypeStruct((VOC, PAD), jnp.float32),
    grid=((VOC + VBLK - 1) // VBLK,),
    in_specs=[pl.BlockSpec((DIM, VBLK), lambda i: (0, i))],
    out_specs=pl.BlockSpec((VBLK, PAD), lambda i: (i, 0)),
)
def _tc_pad(tt_ref, o_ref):
    o_ref[:, :DIM] = lax.dot_general(
        tt_ref[...], jnp.eye(DIM, dtype=jnp.float32),
        (((0,), (0,)), ((), ())),
        preferred_element_type=jnp.float32)


@functools.partial(
    pl.kernel,
    out_type=jax.ShapeDtypeStruct((COLS, DIM, ROWS), jnp.float32),
    mesh=plsc.VectorSubcoreMesh(core_axis_name="c", subcore_axis_name="s"),
    scratch_types=(
        [pltpu.VMEM((PER_W,), jnp.int32)]
        + [pltpu.VMEM((CHB, PAD), jnp.float32) for _ in range(NBUF)]
        + [pltpu.VMEM((DIM, CHB), jnp.float32)]
        + [pltpu.SemaphoreType.DMA for _ in range(NBUF)]
        + [pltpu.SemaphoreType.DMA]
    ),
    compiler_params=pltpu.CompilerParams(
        use_tc_tiling_on_sc=True, needs_layout_passes=False,
        disable_bounds_checks=True),
)
def _sc_gather(xt_hbm, tp_hbm, out_hbm, idx_v, *bufs):
    wid = lax.axis_index("s") * NC + lax.axis_index("c")
    b0 = wid * BPW

    wide = bufs[:NBUF]
    comp = bufs[NBUF]
    sems = bufs[NBUF + 1:2 * NBUF + 1]
    isem = bufs[2 * NBUF + 1]

    for s in range(COLS):
        pltpu.async_copy(
            xt_hbm.at[s, pl.ds(b0, BPW)],
            idx_v.at[pl.ds(s * BPW, BPW)], isem)
    for s in range(COLS):
        pltpu.make_async_copy(
            xt_hbm.at[s, pl.ds(b0, BPW)],
            idx_v.at[pl.ds(s * BPW, BPW)], isem).wait()

    iota = lax.iota(jnp.int32, 16)

    def start_gather(c, b):
        pltpu.async_copy(
            tp_hbm.at[idx_v.at[pl.ds(c * CHB, CHB)]], wide[b], sems[b])

    def wait_gather(c, b):
        pltpu.make_async_copy(
            tp_hbm.at[idx_v.at[pl.ds(c * CHB, CHB)]], wide[b],
            sems[b]).wait()

    def extract(b):
        def grp(g, carry):
            kvec = g * 16 + iota
            for d in range(DIM):
                val = plsc.load_gather(
                    wide[b], [kvec, jnp.full((16,), d, jnp.int32)])
                comp[d, pl.ds(g * 16, 16)] = val
            return carry
        lax.fori_loop(0, CHB // 16, grp, 0)

    def writeback(c, b):
        s = c >> 1
        bb = b0 + (c & 1) * CHB
        pltpu.sync_copy(comp, out_hbm.at[s, :, pl.ds(bb, CHB)])

    def step(c, b):
        wait_gather(c, b)
        extract(b)
        writeback(c, b)

    for b in range(NBUF):
        start_gather(b, b)

    def body(i, carry):
        for b in range(NBUF):
            c = i * NBUF + b
            step(c, b)
            start_gather(c + NBUF, b)
        return carry

    steady = (NCHUNK - NBUF) // NBUF
    lax.fori_loop(0, steady, body, 0)
    for c in range(steady * NBUF, NCHUNK):
        step(c, c % NBUF)
        if c + NBUF < NCHUNK:
            start_gather(c + NBUF, c % NBUF)


def kernel(x, table):
    tp = _tc_pad(table.T)
    out = _sc_gather(x.T, tp)
    return out.transpose(2, 0, 1)
